# baseline (speedup 1.0000x reference)
import functools

import jax
import jax.numpy as jnp
from jax import lax
from jax.experimental import pallas as pl
from jax.experimental.pallas import tpu as pltpu

H = 128
W = 128
HW = H * W
PAD = 3 * W
CH = 1024
NCHUNK = HW // CH
BF = jnp.bfloat16


def _dw_chunk(pad_ref, C, base, kvec, wl):
    P = [None] * 5
    for dh in range(-2, 3):
        xv = pad_ref[:C, pl.ds(base + PAD + (dh - 1) * W, CH + 2 * W)]
        for dwi in range(5):
            t = (dh + 2) * 5 + dwi
            contrib = xv * lax.slice(kvec, (0, t), (C, t + 1))
            P[dwi] = contrib if P[dwi] is None else P[dwi] + contrib
    acc = lax.slice(P[2], (0, W), (C, W + CH))
    zero = jnp.zeros((), BF)
    for dwi, dw in ((0, -2), (1, -1), (3, 1), (4, 2)):
        term = lax.slice(P[dwi], (0, W + dw), (C, W + dw + CH))
        if dw > 0:
            term = jnp.where(wl < (W - dw), term, zero)
        else:
            term = jnp.where(wl >= (-dw), term, zero)
        acc = acc + term
    return jnp.clip(acc, zero, jnp.asarray(6.0, BF))


def _mm(w, a):
    return jnp.dot(w, a, preferred_element_type=jnp.float32)


def _stage1(src_ref, dst_ref, C, kvec, pw, pb, wl):
    def body(j, carry):
        base = j * CH
        a = _dw_chunk(src_ref, C, base, kvec, wl)
        h = jnp.clip(_mm(pw, a) + pb, 0.0, 6.0)
        dst_ref[:pw.shape[0], pl.ds(base + PAD, CH)] = h.astype(BF)
        return carry
    lax.fori_loop(0, NCHUNK, body, 0, unroll=False)


def _stage2(src_ref, out_ref, C, kvec, pw, pb, hw_, hb, wl, post):
    def body(j, carry):
        base = j * CH
        a = _dw_chunk(src_ref, C, base, kvec, wl)
        h = jnp.clip(_mm(pw, a) + pb, 0.0, 6.0)
        o = _mm(hw_, h.astype(BF)) + hb
        out_ref[0, :, pl.ds(base, CH)] = post(o)
        return carry
    lax.fori_loop(0, NCHUNK, body, 0, unroll=False)


def _ttf_kernel(x_ref, hm_k0, hm_w0, hm_b0, hm_k1, hm_w1, hm_b1, hm_hw, hm_hb,
                wh_k0, wh_w0, wh_b0, wh_k1, wh_w1, wh_b1, wh_hw, wh_hb,
                hm_out, reg_out, pada, padb):
    @pl.when(pl.program_id(0) == 0)
    def _init():
        pada[:, :PAD] = jnp.zeros((pada.shape[0], PAD), BF)
        pada[:, PAD + HW:] = jnp.zeros((pada.shape[0], PAD), BF)
        padb[:, :PAD] = jnp.zeros((padb.shape[0], PAD), BF)
        padb[:, PAD + HW:] = jnp.zeros((padb.shape[0], PAD), BF)

    wl = lax.broadcasted_iota(jnp.int32, (1, CH), 1) % W
    Cin = x_ref.shape[1]

    pada[:Cin, PAD:PAD + HW] = x_ref[0].astype(BF)

    _stage1(pada, padb, Cin, hm_k0[...], hm_w0[...], hm_b0[...], wl)
    _stage2(padb, hm_out, hm_w0.shape[0], hm_k1[...], hm_w1[...], hm_b1[...],
            hm_hw[...], hm_hb[...], wl, lambda o: o)

    _stage1(pada, padb, Cin, wh_k0[...], wh_w0[...], wh_b0[...], wl)
    _stage2(padb, reg_out, wh_w0.shape[0], wh_k1[...], wh_w1[...], wh_b1[...],
            wh_hw[...], wh_hb[...], wl, lambda o: jax.nn.relu(o) * 16.0)


@jax.jit
def kernel(x, hm_dw0, hm_pw0, hm_pwb0, hm_dw1, hm_pw1, hm_pwb1, hm_head_w,
           hm_head_b, wh_dw0, wh_pw0, wh_pwb0, wh_dw1, wh_pw1, wh_pwb1,
           wh_head_w, wh_head_b):
    B, Cin, _, _ = x.shape
    nc = hm_head_w.shape[0]
    xf = x.reshape(B, Cin, HW)

    hm_k0 = hm_dw0.reshape(Cin, 25).astype(BF)
    hm_w0 = hm_pw0.reshape(hm_pw0.shape[0], Cin).astype(BF)
    hm_b0 = hm_pwb0.reshape(-1, 1)
    hm_k1 = hm_dw1.reshape(hm_dw1.shape[0], 25).astype(BF)
    hm_w1 = hm_pw1.reshape(hm_pw1.shape[0], hm_pw1.shape[1]).astype(BF)
    hm_b1 = hm_pwb1.reshape(-1, 1)
    hm_hw = hm_head_w.reshape(nc, hm_head_w.shape[1]).astype(BF)
    hm_hb = hm_head_b.reshape(-1, 1)
    wh_k0 = wh_dw0.reshape(Cin, 25).astype(BF)
    wh_w0 = wh_pw0.reshape(wh_pw0.shape[0], Cin).astype(BF)
    wh_b0 = wh_pwb0.reshape(-1, 1)
    wh_k1 = wh_dw1.reshape(wh_dw1.shape[0], 25).astype(BF)
    wh_w1 = wh_pw1.reshape(wh_pw1.shape[0], wh_pw1.shape[1]).astype(BF)
    wh_b1 = wh_pwb1.reshape(-1, 1)
    wh_hw = jnp.zeros((8, wh_head_w.shape[1]), BF).at[:4].set(
        wh_head_w.reshape(4, wh_head_w.shape[1]).astype(BF))
    wh_hb = jnp.zeros((8, 1), jnp.float32).at[:4, 0].set(wh_head_b)

    ws = [hm_k0, hm_w0, hm_b0, hm_k1, hm_w1, hm_b1, hm_hw, hm_hb,
          wh_k0, wh_w0, wh_b0, wh_k1, wh_w1, wh_b1, wh_hw, wh_hb]
    rep = lambda a: pl.BlockSpec(a.shape, lambda i: (0, 0))
    hm_f, reg_f = pl.pallas_call(
        _ttf_kernel,
        grid=(B,),
        in_specs=[pl.BlockSpec((1, Cin, HW), lambda i: (i, 0, 0))] +
                 [rep(a) for a in ws],
        out_specs=[
            pl.BlockSpec((1, nc, HW), lambda i: (i, 0, 0)),
            pl.BlockSpec((1, 8, HW), lambda i: (i, 0, 0)),
        ],
        out_shape=[
            jax.ShapeDtypeStruct((B, nc, HW), jnp.float32),
            jax.ShapeDtypeStruct((B, 8, HW), jnp.float32),
        ],
        scratch_shapes=[
            pltpu.VMEM((Cin, HW + 2 * PAD), BF),
            pltpu.VMEM((128, HW + 2 * PAD), BF),
        ],
    )(xf, *ws)

    heatmap = hm_f.reshape(B, nc, H, W)
    reg_box = reg_f[:, :4].reshape(B, 4, H, W)
    return (heatmap, reg_box)


# speedup vs baseline: 1.4286x; 1.4286x over previous
import functools

import jax
import jax.numpy as jnp
from jax import lax
from jax.experimental import pallas as pl
from jax.experimental.pallas import tpu as pltpu

H = 128
W = 128
HW = H * W
PAD = 3 * W
CH = 4096
NCHUNK = HW // CH
BF = jnp.bfloat16


def _dw_chunk(pad_ref, C, base, kvec, wl):
    P = [None] * 5
    for dh in range(-2, 3):
        xv = pad_ref[:C, pl.ds(base + PAD + (dh - 1) * W, CH + 2 * W)]
        for dwi in range(5):
            t = (dh + 2) * 5 + dwi
            contrib = xv * lax.slice(kvec, (0, t), (C, t + 1))
            P[dwi] = contrib if P[dwi] is None else P[dwi] + contrib
    acc = lax.slice(P[2], (0, W), (C, W + CH))
    zero = jnp.zeros((), BF)
    for dwi, dw in ((0, -2), (1, -1), (3, 1), (4, 2)):
        term = lax.slice(P[dwi], (0, W + dw), (C, W + dw + CH))
        if dw > 0:
            term = jnp.where(wl < (W - dw), term, zero)
        else:
            term = jnp.where(wl >= (-dw), term, zero)
        acc = acc + term
    return jnp.clip(acc, zero, jnp.asarray(6.0, BF))


def _mm(w, a):
    return jnp.dot(w, a, preferred_element_type=jnp.float32)


def _stage1(src_ref, dst_ref, C, kvec, pw, pb, wl):
    def body(j, carry):
        base = j * CH
        a = _dw_chunk(src_ref, C, base, kvec, wl)
        h = jnp.clip(_mm(pw, a) + pb, 0.0, 6.0)
        dst_ref[:pw.shape[0], pl.ds(base + PAD, CH)] = h.astype(BF)
        return carry
    lax.fori_loop(0, NCHUNK, body, 0, unroll=False)


def _stage2(src_ref, out_ref, C, kvec, pw, pb, hw_, hb, wl, post):
    def body(j, carry):
        base = j * CH
        a = _dw_chunk(src_ref, C, base, kvec, wl)
        h = jnp.clip(_mm(pw, a) + pb, 0.0, 6.0)
        o = _mm(hw_, h.astype(BF)) + hb
        out_ref[0, :, pl.ds(base, CH)] = post(o)
        return carry
    lax.fori_loop(0, NCHUNK, body, 0, unroll=False)


def _ttf_kernel(x_ref, hm_k0, hm_w0, hm_b0, hm_k1, hm_w1, hm_b1, hm_hw, hm_hb,
                wh_k0, wh_w0, wh_b0, wh_k1, wh_w1, wh_b1, wh_hw, wh_hb,
                hm_out, reg_out, pada, padb):
    @pl.when(pl.program_id(0) == 0)
    def _init():
        pada[:, :PAD] = jnp.zeros((pada.shape[0], PAD), BF)
        pada[:, PAD + HW:] = jnp.zeros((pada.shape[0], PAD), BF)
        padb[:, :PAD] = jnp.zeros((padb.shape[0], PAD), BF)
        padb[:, PAD + HW:] = jnp.zeros((padb.shape[0], PAD), BF)

    wl = lax.broadcasted_iota(jnp.int32, (1, CH), 1) % W
    Cin = x_ref.shape[1]

    pada[:Cin, PAD:PAD + HW] = x_ref[0].astype(BF)

    _stage1(pada, padb, Cin, hm_k0[...], hm_w0[...], hm_b0[...], wl)
    _stage2(padb, hm_out, hm_w0.shape[0], hm_k1[...], hm_w1[...], hm_b1[...],
            hm_hw[...], hm_hb[...], wl, lambda o: o)

    _stage1(pada, padb, Cin, wh_k0[...], wh_w0[...], wh_b0[...], wl)
    _stage2(padb, reg_out, wh_w0.shape[0], wh_k1[...], wh_w1[...], wh_b1[...],
            wh_hw[...], wh_hb[...], wl, lambda o: jax.nn.relu(o) * 16.0)


@jax.jit
def kernel(x, hm_dw0, hm_pw0, hm_pwb0, hm_dw1, hm_pw1, hm_pwb1, hm_head_w,
           hm_head_b, wh_dw0, wh_pw0, wh_pwb0, wh_dw1, wh_pw1, wh_pwb1,
           wh_head_w, wh_head_b):
    B, Cin, _, _ = x.shape
    nc = hm_head_w.shape[0]
    xf = x.reshape(B, Cin, HW)

    hm_k0 = hm_dw0.reshape(Cin, 25).astype(BF)
    hm_w0 = hm_pw0.reshape(hm_pw0.shape[0], Cin).astype(BF)
    hm_b0 = hm_pwb0.reshape(-1, 1)
    hm_k1 = hm_dw1.reshape(hm_dw1.shape[0], 25).astype(BF)
    hm_w1 = hm_pw1.reshape(hm_pw1.shape[0], hm_pw1.shape[1]).astype(BF)
    hm_b1 = hm_pwb1.reshape(-1, 1)
    hm_hw = hm_head_w.reshape(nc, hm_head_w.shape[1]).astype(BF)
    hm_hb = hm_head_b.reshape(-1, 1)
    wh_k0 = wh_dw0.reshape(Cin, 25).astype(BF)
    wh_w0 = wh_pw0.reshape(wh_pw0.shape[0], Cin).astype(BF)
    wh_b0 = wh_pwb0.reshape(-1, 1)
    wh_k1 = wh_dw1.reshape(wh_dw1.shape[0], 25).astype(BF)
    wh_w1 = wh_pw1.reshape(wh_pw1.shape[0], wh_pw1.shape[1]).astype(BF)
    wh_b1 = wh_pwb1.reshape(-1, 1)
    wh_hw = jnp.zeros((8, wh_head_w.shape[1]), BF).at[:4].set(
        wh_head_w.reshape(4, wh_head_w.shape[1]).astype(BF))
    wh_hb = jnp.zeros((8, 1), jnp.float32).at[:4, 0].set(wh_head_b)

    ws = [hm_k0, hm_w0, hm_b0, hm_k1, hm_w1, hm_b1, hm_hw, hm_hb,
          wh_k0, wh_w0, wh_b0, wh_k1, wh_w1, wh_b1, wh_hw, wh_hb]
    rep = lambda a: pl.BlockSpec(a.shape, lambda i: (0, 0))
    hm_f, reg_f = pl.pallas_call(
        _ttf_kernel,
        grid=(B,),
        in_specs=[pl.BlockSpec((1, Cin, HW), lambda i: (i, 0, 0))] +
                 [rep(a) for a in ws],
        out_specs=[
            pl.BlockSpec((1, nc, HW), lambda i: (i, 0, 0)),
            pl.BlockSpec((1, 8, HW), lambda i: (i, 0, 0)),
        ],
        out_shape=[
            jax.ShapeDtypeStruct((B, nc, HW), jnp.float32),
            jax.ShapeDtypeStruct((B, 8, HW), jnp.float32),
        ],
        scratch_shapes=[
            pltpu.VMEM((Cin, HW + 2 * PAD), BF),
            pltpu.VMEM((128, HW + 2 * PAD), BF),
        ],
    )(xf, *ws)

    heatmap = hm_f.reshape(B, nc, H, W)
    reg_box = reg_f[:, :4].reshape(B, 4, H, W)
    return (heatmap, reg_box)


# speedup vs baseline: 1.4914x; 1.0440x over previous
import functools

import jax
import jax.numpy as jnp
from jax import lax
from jax.experimental import pallas as pl
from jax.experimental.pallas import tpu as pltpu

H = 128
W = 128
HW = H * W
PAD = 3 * W
CH = 8192
NCHUNK = HW // CH
BF = jnp.bfloat16


def _dw_chunk(pad_ref, C, base, kvec, wl):
    P = [None] * 5
    for dh in range(-2, 3):
        xv = pad_ref[:C, pl.ds(base + PAD + (dh - 1) * W, CH + 2 * W)]
        for dwi in range(5):
            t = (dh + 2) * 5 + dwi
            contrib = xv * lax.slice(kvec, (0, t), (C, t + 1))
            P[dwi] = contrib if P[dwi] is None else P[dwi] + contrib
    acc = lax.slice(P[2], (0, W), (C, W + CH))
    zero = jnp.zeros((), BF)
    for dwi, dw in ((0, -2), (1, -1), (3, 1), (4, 2)):
        term = lax.slice(P[dwi], (0, W + dw), (C, W + dw + CH))
        if dw > 0:
            term = jnp.where(wl < (W - dw), term, zero)
        else:
            term = jnp.where(wl >= (-dw), term, zero)
        acc = acc + term
    return jnp.clip(acc, zero, jnp.asarray(6.0, BF))


def _mm(w, a):
    return jnp.dot(w, a, preferred_element_type=jnp.float32)


def _stage1(src_ref, dst_ref, C, kvec, pw, pb, wl):
    def body(j, carry):
        base = j * CH
        a = _dw_chunk(src_ref, C, base, kvec, wl)
        h = jnp.clip(_mm(pw, a) + pb, 0.0, 6.0)
        dst_ref[:pw.shape[0], pl.ds(base + PAD, CH)] = h.astype(BF)
        return carry
    lax.fori_loop(0, NCHUNK, body, 0, unroll=False)


def _stage2(src_ref, out_ref, C, kvec, pw, pb, hw_, hb, wl, post):
    def body(j, carry):
        base = j * CH
        a = _dw_chunk(src_ref, C, base, kvec, wl)
        h = jnp.clip(_mm(pw, a) + pb, 0.0, 6.0)
        o = _mm(hw_, h.astype(BF)) + hb
        out_ref[0, :, pl.ds(base, CH)] = post(o)
        return carry
    lax.fori_loop(0, NCHUNK, body, 0, unroll=False)


def _ttf_kernel(x_ref, hm_k0, hm_w0, hm_b0, hm_k1, hm_w1, hm_b1, hm_hw, hm_hb,
                wh_k0, wh_w0, wh_b0, wh_k1, wh_w1, wh_b1, wh_hw, wh_hb,
                hm_out, reg_out, pada, padb):
    @pl.when(pl.program_id(0) == 0)
    def _init():
        pada[:, :PAD] = jnp.zeros((pada.shape[0], PAD), BF)
        pada[:, PAD + HW:] = jnp.zeros((pada.shape[0], PAD), BF)
        padb[:, :PAD] = jnp.zeros((padb.shape[0], PAD), BF)
        padb[:, PAD + HW:] = jnp.zeros((padb.shape[0], PAD), BF)

    wl = lax.broadcasted_iota(jnp.int32, (1, CH), 1) % W
    Cin = x_ref.shape[1]

    pada[:Cin, PAD:PAD + HW] = x_ref[0].astype(BF)

    _stage1(pada, padb, Cin, hm_k0[...], hm_w0[...], hm_b0[...], wl)
    _stage2(padb, hm_out, hm_w0.shape[0], hm_k1[...], hm_w1[...], hm_b1[...],
            hm_hw[...], hm_hb[...], wl, lambda o: o)

    _stage1(pada, padb, Cin, wh_k0[...], wh_w0[...], wh_b0[...], wl)
    _stage2(padb, reg_out, wh_w0.shape[0], wh_k1[...], wh_w1[...], wh_b1[...],
            wh_hw[...], wh_hb[...], wl, lambda o: jax.nn.relu(o) * 16.0)


@jax.jit
def kernel(x, hm_dw0, hm_pw0, hm_pwb0, hm_dw1, hm_pw1, hm_pwb1, hm_head_w,
           hm_head_b, wh_dw0, wh_pw0, wh_pwb0, wh_dw1, wh_pw1, wh_pwb1,
           wh_head_w, wh_head_b):
    B, Cin, _, _ = x.shape
    nc = hm_head_w.shape[0]
    xf = x.reshape(B, Cin, HW)

    hm_k0 = hm_dw0.reshape(Cin, 25).astype(BF)
    hm_w0 = hm_pw0.reshape(hm_pw0.shape[0], Cin).astype(BF)
    hm_b0 = hm_pwb0.reshape(-1, 1)
    hm_k1 = hm_dw1.reshape(hm_dw1.shape[0], 25).astype(BF)
    hm_w1 = hm_pw1.reshape(hm_pw1.shape[0], hm_pw1.shape[1]).astype(BF)
    hm_b1 = hm_pwb1.reshape(-1, 1)
    hm_hw = hm_head_w.reshape(nc, hm_head_w.shape[1]).astype(BF)
    hm_hb = hm_head_b.reshape(-1, 1)
    wh_k0 = wh_dw0.reshape(Cin, 25).astype(BF)
    wh_w0 = wh_pw0.reshape(wh_pw0.shape[0], Cin).astype(BF)
    wh_b0 = wh_pwb0.reshape(-1, 1)
    wh_k1 = wh_dw1.reshape(wh_dw1.shape[0], 25).astype(BF)
    wh_w1 = wh_pw1.reshape(wh_pw1.shape[0], wh_pw1.shape[1]).astype(BF)
    wh_b1 = wh_pwb1.reshape(-1, 1)
    wh_hw = jnp.zeros((8, wh_head_w.shape[1]), BF).at[:4].set(
        wh_head_w.reshape(4, wh_head_w.shape[1]).astype(BF))
    wh_hb = jnp.zeros((8, 1), jnp.float32).at[:4, 0].set(wh_head_b)

    ws = [hm_k0, hm_w0, hm_b0, hm_k1, hm_w1, hm_b1, hm_hw, hm_hb,
          wh_k0, wh_w0, wh_b0, wh_k1, wh_w1, wh_b1, wh_hw, wh_hb]
    rep = lambda a: pl.BlockSpec(a.shape, lambda i: (0, 0))
    hm_f, reg_f = pl.pallas_call(
        _ttf_kernel,
        grid=(B,),
        in_specs=[pl.BlockSpec((1, Cin, HW), lambda i: (i, 0, 0))] +
                 [rep(a) for a in ws],
        out_specs=[
            pl.BlockSpec((1, nc, HW), lambda i: (i, 0, 0)),
            pl.BlockSpec((1, 8, HW), lambda i: (i, 0, 0)),
        ],
        out_shape=[
            jax.ShapeDtypeStruct((B, nc, HW), jnp.float32),
            jax.ShapeDtypeStruct((B, 8, HW), jnp.float32),
        ],
        scratch_shapes=[
            pltpu.VMEM((Cin, HW + 2 * PAD), BF),
            pltpu.VMEM((128, HW + 2 * PAD), BF),
        ],
    )(xf, *ws)

    heatmap = hm_f.reshape(B, nc, H, W)
    reg_box = reg_f[:, :4].reshape(B, 4, H, W)
    return (heatmap, reg_box)
